# baseline (speedup 1.0000x reference)
import numpy as np
import jax
import jax.numpy as jnp
from jax import lax
from jax.experimental import pallas as pl
from jax.experimental.pallas import tpu as pltpu
from jax.experimental.pallas import tpu_sc as plsc

HW = 16384
N = 2048
BOUND = np.float32(0.01 * 2048)
WLAST = np.float32(0.01 * 2048 - 21.0)
INV_BOUND = np.float32(1.0 / (0.01 * 2048))
INF = np.float32(np.inf)
L = 16
NW = 32
QPW = HW // NW
NCH = N // L
U = 8
RCAP = 128
SEGS = ((0, 1), (1, 2), (2, 4), (4, 8), (8, 16))


GSTEP = np.float32(2.0 / 127.0)


def _dtm_body(xx_hbm, xy_hbm, out_hbm, px_ref, py_ref,
              c0, c1, c2, c3, c4, c5, c6, c7, out_ref):
    cand = (c0, c1, c2, c3, c4, c5, c6, c7)
    wid = lax.axis_index("s") * 2 + lax.axis_index("c")
    qbase = wid * QPW
    pltpu.sync_copy(xx_hbm, px_ref)
    pltpu.sync_copy(xy_hbm, py_ref)

    iota = lax.iota(jnp.int32, L)
    inf_v = jnp.full((L,), INF, jnp.float32)

    def merge3(blo, bhi, csort):
        r = jnp.flip(csort)
        l1 = jnp.minimum(bhi, r)
        r2 = jnp.flip(jnp.sort(l1))
        l2 = jnp.minimum(blo, r2)
        h2 = jnp.maximum(blo, r2)
        return jnp.sort(l2), jnp.sort(h2)

    def chunk_append(j, u, qx, qy, t21, ptr_u):
        px = px_ref[pl.ds(j * L, L)]
        py = py_ref[pl.ds(j * L, L)]
        dx = px - qx
        dy = py - qy
        d = dx * dx + dy * dy
        mask = d < t21
        pc = plsc.cumsum(mask.astype(jnp.int32))
        idx = pc + ptr_u - 1
        plsc.store_scatter(cand[u], [idx], d, mask=mask)
        return ptr_u + plsc.all_reduce_population_count(mask)

    def drain_all(blo, bhi, ptrs):
        for u in range(U):
            p_u = ptrs[u][0]
            def sub(i, b, u=u, p_u=p_u):
                c = cand[u][pl.ds(i * L, L)]
                c = jnp.where(iota < (p_u - i * L), c, INF)
                return merge3(b[0], b[1], jnp.sort(c))
            nsub = lax.shift_right_logical(p_u + (L - 1), 4)
            blo, bhi = lax.fori_loop(0, nsub, sub, (blo, bhi))
        t21 = jnp.full((L,), bhi[4])
        return blo, bhi, t21

    def group_body(g, _):
        def query_body(l, outacc):
            q = qbase + g * L + l
            col = (q & 127).astype(jnp.float32)
            row = lax.shift_right_logical(q, 7).astype(jnp.float32)
            qx = jnp.full((L,), col * GSTEP - 1.0, jnp.float32)
            qy = jnp.full((L,), 1.0 - row * GSTEP, jnp.float32)
            blo, bhi, t21 = inf_v, inf_v, inf_v
            zeros = (jnp.zeros((L,), jnp.int32),) * U
            ptrs = zeros
            for (lo, hi) in SEGS:
                def seg_body(it, p, qx=qx, qy=qy, t21=t21):
                    return tuple(
                        chunk_append(it * U + u, u, qx, qy, t21, p[u])
                        for u in range(U))
                ptrs = lax.fori_loop(lo, hi, seg_body, ptrs)
                blo, bhi, t21 = drain_all(blo, bhi, ptrs)
                ptrs = zeros
            s16 = jnp.sum(blo)
            s5 = jnp.sum(jnp.where(iota < 5, bhi, jnp.float32(0.0)))
            val = (s16 + s5 + bhi[4] * WLAST) * INV_BOUND
            return jnp.where(iota == l, val, outacc)

        outacc = lax.fori_loop(0, L, query_body, inf_v)
        bits = lax.bitcast_convert_type(outacc, jnp.int32)
        y = lax.bitcast_convert_type(
            lax.shift_right_arithmetic(bits, 1) + 0x1FBD1DF5, jnp.float32)
        for _ in range(3):
            y = 0.5 * (y + outacc / y)
        out_ref[pl.ds(g * L, L)] = y
        return 0

    lax.fori_loop(0, QPW // L, group_body, 0)
    pltpu.sync_copy(out_ref, out_hbm.at[pl.ds(qbase, QPW)])


_dtm = pl.kernel(
    _dtm_body,
    out_type=jax.ShapeDtypeStruct((HW,), jnp.float32),
    mesh=plsc.VectorSubcoreMesh(core_axis_name="c", subcore_axis_name="s"),
    compiler_params=pltpu.CompilerParams(needs_layout_passes=False),
    scratch_types=[
        pltpu.VMEM((N,), jnp.float32),
        pltpu.VMEM((N,), jnp.float32),
        pltpu.VMEM((RCAP,), jnp.float32),
        pltpu.VMEM((RCAP,), jnp.float32),
        pltpu.VMEM((RCAP,), jnp.float32),
        pltpu.VMEM((RCAP,), jnp.float32),
        pltpu.VMEM((RCAP,), jnp.float32),
        pltpu.VMEM((RCAP,), jnp.float32),
        pltpu.VMEM((RCAP,), jnp.float32),
        pltpu.VMEM((RCAP,), jnp.float32),
        pltpu.VMEM((QPW,), jnp.float32),
    ],
)


def kernel(x):
    return _dtm(x[:, 0], x[:, 1]).reshape(128, 128)


# speedup vs baseline: 1.3897x; 1.3897x over previous
import numpy as np
import jax
import jax.numpy as jnp
from jax import lax
from jax.experimental import pallas as pl
from jax.experimental.pallas import tpu as pltpu
from jax.experimental.pallas import tpu_sc as plsc

HW = 16384
N = 2048
BOUND = np.float32(0.01 * 2048)
WLAST = np.float32(0.01 * 2048 - 21.0)
INV_BOUND = np.float32(1.0 / (0.01 * 2048))
INF = np.float32(np.inf)
L = 16
NW = 32
QPW = HW // NW
NCH = N // L
U = 8
RCAP = 128
SEGS = ((0, 1), (1, 2), (2, 4), (4, 8), (8, 16))


GSTEP = np.float32(2.0 / 127.0)


def _dtm_body(xx_hbm, xy_hbm, out_hbm, px_ref, py_ref,
              c0, c1, c2, c3, c4, c5, c6, c7, out_ref):
    cand = (c0, c1, c2, c3, c4, c5, c6, c7)
    wid = lax.axis_index("s") * 2 + lax.axis_index("c")
    qbase = wid * QPW
    pltpu.sync_copy(xx_hbm, px_ref)
    pltpu.sync_copy(xy_hbm, py_ref)

    iota = lax.iota(jnp.int32, L)
    inf_v = jnp.full((L,), INF, jnp.float32)

    def merge3(blo, bhi, csort):
        r = jnp.flip(csort)
        l1 = jnp.minimum(bhi, r)
        r2 = jnp.flip(jnp.sort(l1))
        l2 = jnp.minimum(blo, r2)
        h2 = jnp.maximum(blo, r2)
        return jnp.sort(l2), jnp.sort(h2)

    def chunks_append(it, qx, qy, t21i, ptrs):
        pxs = [px_ref[pl.ds((it * U + u) * L, L)] for u in range(U)]
        pys = [py_ref[pl.ds((it * U + u) * L, L)] for u in range(U)]
        dxs = [pxs[u] - qx for u in range(U)]
        dys = [pys[u] - qy for u in range(U)]
        ds = [dxs[u] * dxs[u] + dys[u] * dys[u] for u in range(U)]
        dbits = [lax.bitcast_convert_type(ds[u], jnp.int32) for u in range(U)]
        masks = [dbits[u] < t21i for u in range(U)]
        pcs = [plsc.cumsum(masks[u].astype(jnp.int32)) for u in range(U)]
        new_ptrs = []
        for u in range(U):
            idx = pcs[u] + ptrs[u] - 1
            plsc.store_scatter(cand[u], [idx], ds[u], mask=masks[u])
            new_ptrs.append(ptrs[u] + plsc.all_reduce_population_count(masks[u]))
        return tuple(new_ptrs)

    def drain_all(blo, bhi, ptrs):
        for u in range(U):
            p_u = ptrs[u][0]
            def sub(i, b, u=u, p_u=p_u):
                c = cand[u][pl.ds(i * L, L)]
                c = jnp.where(iota < (p_u - i * L), c, INF)
                return merge3(b[0], b[1], jnp.sort(c))
            nsub = lax.shift_right_logical(p_u + (L - 1), 4)
            blo, bhi = lax.fori_loop(0, nsub, sub, (blo, bhi))
        t21i = lax.bitcast_convert_type(jnp.full((L,), bhi[4]), jnp.int32)
        return blo, bhi, t21i

    def group_body(g, _):
        def query_body(l, outacc):
            q = qbase + g * L + l
            col = (q & 127).astype(jnp.float32)
            row = lax.shift_right_logical(q, 7).astype(jnp.float32)
            qx = jnp.full((L,), col * GSTEP - 1.0, jnp.float32)
            qy = jnp.full((L,), 1.0 - row * GSTEP, jnp.float32)
            blo, bhi = inf_v, inf_v
            t21i = lax.bitcast_convert_type(inf_v, jnp.int32)
            zeros = (jnp.zeros((L,), jnp.int32),) * U
            ptrs = zeros
            for (lo, hi) in SEGS:
                def seg_body(it, p, qx=qx, qy=qy, t21i=t21i):
                    return chunks_append(it, qx, qy, t21i, p)
                ptrs = lax.fori_loop(lo, hi, seg_body, ptrs)
                blo, bhi, t21i = drain_all(blo, bhi, ptrs)
                ptrs = zeros
            s16 = jnp.sum(blo)
            s5 = jnp.sum(jnp.where(iota < 5, bhi, jnp.float32(0.0)))
            val = (s16 + s5 + bhi[4] * WLAST) * INV_BOUND
            return jnp.where(iota == l, val, outacc)

        outacc = lax.fori_loop(0, L, query_body, inf_v)
        bits = lax.bitcast_convert_type(outacc, jnp.int32)
        y = lax.bitcast_convert_type(
            lax.shift_right_arithmetic(bits, 1) + 0x1FBD1DF5, jnp.float32)
        for _ in range(3):
            y = 0.5 * (y + outacc / y)
        out_ref[pl.ds(g * L, L)] = y
        return 0

    lax.fori_loop(0, QPW // L, group_body, 0)
    pltpu.sync_copy(out_ref, out_hbm.at[pl.ds(qbase, QPW)])


_dtm = pl.kernel(
    _dtm_body,
    out_type=jax.ShapeDtypeStruct((HW,), jnp.float32),
    mesh=plsc.VectorSubcoreMesh(core_axis_name="c", subcore_axis_name="s"),
    compiler_params=pltpu.CompilerParams(needs_layout_passes=False),
    scratch_types=[
        pltpu.VMEM((N,), jnp.float32),
        pltpu.VMEM((N,), jnp.float32),
        pltpu.VMEM((RCAP,), jnp.float32),
        pltpu.VMEM((RCAP,), jnp.float32),
        pltpu.VMEM((RCAP,), jnp.float32),
        pltpu.VMEM((RCAP,), jnp.float32),
        pltpu.VMEM((RCAP,), jnp.float32),
        pltpu.VMEM((RCAP,), jnp.float32),
        pltpu.VMEM((RCAP,), jnp.float32),
        pltpu.VMEM((RCAP,), jnp.float32),
        pltpu.VMEM((QPW,), jnp.float32),
    ],
)


def kernel(x):
    return _dtm(x[:, 0], x[:, 1]).reshape(128, 128)


# speedup vs baseline: 2.7388x; 1.9708x over previous
import numpy as np
import jax
import jax.numpy as jnp
from jax import lax
from jax.experimental import pallas as pl
from jax.experimental.pallas import tpu as pltpu
from jax.experimental.pallas import tpu_sc as plsc

HW = 16384
N = 2048
BOUND = np.float32(0.01 * 2048)
WLAST = np.float32(0.01 * 2048 - 21.0)
INV_BOUND = np.float32(1.0 / (0.01 * 2048))
INF = np.float32(np.inf)
L = 16
NW = 32
QPW = HW // NW
NCH = N // L
U = 8
RCAP = 144
SEGS = ((0, 1), (1, 3), (3, 7), (7, 16))
GSTEP = np.float32(2.0 / 127.0)


def _dtm_body(xx_hbm, xy_hbm, out_hbm, px_ref, py_ref,
              c0, c1, c2, c3, c4, c5, c6, c7,
              c8, c9, c10, c11, c12, c13, c14, c15, out_ref):
    cand = (c0, c1, c2, c3, c4, c5, c6, c7,
            c8, c9, c10, c11, c12, c13, c14, c15)
    wid = lax.axis_index("s") * 2 + lax.axis_index("c")
    qbase = wid * QPW
    pltpu.sync_copy(xx_hbm, px_ref)
    pltpu.sync_copy(xy_hbm, py_ref)

    iota = lax.iota(jnp.int32, L)
    inf_v = jnp.full((L,), INF, jnp.float32)
    infbits_v = lax.bitcast_convert_type(inf_v, jnp.int32)

    def merge3(blo, bhi, csort):
        r = jnp.flip(csort)
        l1 = jnp.minimum(bhi, r)
        r2 = jnp.flip(jnp.sort(l1))
        l2 = jnp.minimum(blo, r2)
        h2 = jnp.maximum(blo, r2)
        return jnp.sort(l2), jnp.sort(h2)

    def splat_q(q):
        col = (q & 127).astype(jnp.float32)
        row = lax.shift_right_logical(q, 7).astype(jnp.float32)
        qx = jnp.full((L,), col * GSTEP - 1.0, jnp.float32)
        qy = jnp.full((L,), 1.0 - row * GSTEP, jnp.float32)
        return qx, qy

    def chunks_append(it, qs, ts, ptrs):
        (qxa, qya), (qxb, qyb) = qs
        t21ia, t21ib = ts
        pxs = [px_ref[pl.ds((it * U + u) * L, L)] for u in range(U)]
        pys = [py_ref[pl.ds((it * U + u) * L, L)] for u in range(U)]
        dxa = [pxs[u] - qxa for u in range(U)]
        dya = [pys[u] - qya for u in range(U)]
        dxb = [pxs[u] - qxb for u in range(U)]
        dyb = [pys[u] - qyb for u in range(U)]
        da = [dxa[u] * dxa[u] + dya[u] * dya[u] for u in range(U)]
        db = [dxb[u] * dxb[u] + dyb[u] * dyb[u] for u in range(U)]
        dba = [lax.bitcast_convert_type(da[u], jnp.int32) for u in range(U)]
        dbb = [lax.bitcast_convert_type(db[u], jnp.int32) for u in range(U)]
        ma = [dba[u] < t21ia for u in range(U)]
        mb = [dbb[u] < t21ib for u in range(U)]
        pa = [plsc.cumsum(ma[u].astype(jnp.int32)) for u in range(U)]
        pb = [plsc.cumsum(mb[u].astype(jnp.int32)) for u in range(U)]
        out = []
        for u in range(U):
            plsc.store_scatter(cand[u], [pa[u] + (ptrs[u] - 1)],
                               da[u], mask=ma[u])
            out.append(ptrs[u] + pa[u][L - 1])
        for u in range(U):
            plsc.store_scatter(cand[U + u], [pb[u] + (ptrs[U + u] - 1)],
                               db[u], mask=mb[u])
            out.append(ptrs[U + u] + pb[u][L - 1])
        return tuple(out)

    def drain_both(ba, bb, ptrs):
        bloa, bhia = ba
        blob, bhib = bb
        for u in range(U):
            p_a = ptrs[u]
            p_b = ptrs[U + u]
            nsub = lax.max(lax.shift_right_logical(p_a + (L - 1), 4),
                           lax.shift_right_logical(p_b + (L - 1), 4))

            def sub(i, b, u=u, p_a=p_a, p_b=p_b):
                xa, ya, xb, yb = b
                ca = cand[u][pl.ds(i * L, L)]
                cb = cand[U + u][pl.ds(i * L, L)]
                ca = jnp.where(iota < (p_a - i * L), ca, INF)
                cb = jnp.where(iota < (p_b - i * L), cb, INF)
                xa, ya = merge3(xa, ya, jnp.sort(ca))
                xb, yb = merge3(xb, yb, jnp.sort(cb))
                return (xa, ya, xb, yb)

            bloa, bhia, blob, bhib = lax.fori_loop(
                0, nsub, sub, (bloa, bhia, blob, bhib))
        ta = lax.bitcast_convert_type(bhia[4], jnp.int32)
        tb = lax.bitcast_convert_type(bhib[4], jnp.int32)
        return (bloa, bhia), (blob, bhib), (ta, tb)

    def extract(blo, bhi):
        s16 = jnp.sum(blo)
        s5 = jnp.sum(jnp.where(iota < 5, bhi, jnp.float32(0.0)))
        return (s16 + s5 + bhi[4] * WLAST) * INV_BOUND

    def group_body(g, _):
        def pair_body(lp, outacc):
            qa = qbase + g * L + lp
            qs = (splat_q(qa), splat_q(qa + 8))
            ba = (inf_v, inf_v)
            bb = (inf_v, inf_v)
            ts = (jnp.int32(0x7F800000), jnp.int32(0x7F800000))
            zeros = (jnp.int32(0),) * (2 * U)
            ptrs = zeros
            for (lo, hi) in SEGS:
                def seg_body(it, p, qs=qs, ts=ts):
                    return chunks_append(it, qs, ts, p)
                ptrs = lax.fori_loop(lo, hi, seg_body, ptrs)
                ba, bb, ts = drain_both(ba, bb, ptrs)
                ptrs = zeros
            va = extract(*ba)
            vb = extract(*bb)
            outacc = jnp.where(iota == lp, va, outacc)
            return jnp.where(iota == lp + 8, vb, outacc)

        outacc = lax.fori_loop(0, 8, pair_body, inf_v)
        bits = lax.bitcast_convert_type(outacc, jnp.int32)
        y = lax.bitcast_convert_type(
            lax.shift_right_arithmetic(bits, 1) + 0x1FBD1DF5, jnp.float32)
        for _ in range(3):
            y = 0.5 * (y + outacc / y)
        out_ref[pl.ds(g * L, L)] = y
        return 0

    lax.fori_loop(0, QPW // L, group_body, 0)
    pltpu.sync_copy(out_ref, out_hbm.at[pl.ds(qbase, QPW)])


_dtm = pl.kernel(
    _dtm_body,
    out_type=jax.ShapeDtypeStruct((HW,), jnp.float32),
    mesh=plsc.VectorSubcoreMesh(core_axis_name="c", subcore_axis_name="s"),
    compiler_params=pltpu.CompilerParams(needs_layout_passes=False),
    scratch_types=[
        pltpu.VMEM((N,), jnp.float32),
        pltpu.VMEM((N,), jnp.float32),
    ] + [pltpu.VMEM((RCAP,), jnp.float32)] * 16
    + [pltpu.VMEM((QPW,), jnp.float32)],
)


def kernel(x):
    return _dtm(x[:, 0], x[:, 1]).reshape(128, 128)
